# baseline (speedup 1.0000x reference)
import functools

import jax
import jax.numpy as jnp
from jax import lax
from jax.experimental import pallas as pl
from jax.experimental.pallas import tpu as pltpu

T, D, E, RANK, B_DIM = 4096, 2048, 8, 16, 4096
ER = E * RANK
N_OUT = 2 * B_DIM

TN = 512
NJH = B_DIM // TN


def _dot(a, b):
    return lax.dot_general(a.astype(jnp.bfloat16), b.astype(jnp.bfloat16),
                           (((1,), (0,)), ((), ())),
                           preferred_element_type=jnp.float32)


TM1 = 1024


def _xa_kernel(idx_ref, x_ref, ahall_ref, xa_ref):
    xa = _dot(x_ref[...], ahall_ref[...])
    col = jax.lax.broadcasted_iota(jnp.int32, (TM1, 2 * ER), 1)
    col_expert = (col // RANK) % E
    xa = jnp.where(col_expert == idx_ref[...].astype(jnp.int32), xa, 0.0)
    xa_bf = xa.astype(jnp.bfloat16)
    xa_ref[...] = jnp.stack([xa_bf[:, :ER], xa_bf[:, ER:]])


def _main_kernel(x_ref, w_ref, xa_ref, bres_ref, out_ref):
    h = pl.program_id(0) // NJH
    out_ref[...] = _dot(x_ref[...], w_ref[...]) + _dot(xa_ref[h], bres_ref[...])


@functools.partial(jax.jit, static_argnames=())
def kernel(input_, W, A_buffer, B_buffer, weight_indices):
    A_hall = (A_buffer.reshape(E, D, 2, RANK)
              .transpose(1, 2, 0, 3).reshape(D, 2 * ER)).astype(jnp.bfloat16)
    B_res = B_buffer.reshape(ER, N_OUT).astype(jnp.bfloat16)
    x_bf = input_.astype(jnp.bfloat16)
    idx2d = weight_indices.astype(jnp.int8).reshape(T, 1)

    xa = pl.pallas_call(
        _xa_kernel,
        grid=(T // TM1,),
        in_specs=[
            pl.BlockSpec((TM1, 1), lambda i: (i, 0)),
            pl.BlockSpec((TM1, D), lambda i: (i, 0)),
            pl.BlockSpec((D, 2 * ER), lambda i: (0, 0)),
        ],
        out_specs=pl.BlockSpec((2, TM1, ER), lambda i: (0, i, 0)),
        out_shape=jax.ShapeDtypeStruct((2, T, ER), jnp.bfloat16),
    )(idx2d, x_bf, A_hall)

    out = pl.pallas_call(
        _main_kernel,
        grid=(N_OUT // TN,),
        in_specs=[
            pl.BlockSpec((T, D), lambda j: (0, 0)),
            pl.BlockSpec((D, TN), lambda j: (0, j)),
            pl.BlockSpec((2, T, ER), lambda j: (0, 0, 0)),
            pl.BlockSpec((ER, TN), lambda j: (0, j)),
        ],
        out_specs=pl.BlockSpec((T, TN), lambda j: (0, j)),
        out_shape=jax.ShapeDtypeStruct((T, N_OUT), jnp.float32),
    )(x_bf, W, xa, B_res)
    return out
